# baseline (speedup 1.0000x reference)
import jax
import jax.numpy as jnp
from jax.experimental import pallas as pl
from jax.experimental.pallas import tpu as pltpu

SEQ_LEN = 4096
EMB = 1024
CHUNK = 256
NBUF = 16
DEPTH = 6
NCHUNKS = SEQ_LEN // CHUNK


def kernel(input, encoding):
    def body(inp_smem, enc_hbm, out_hbm, buf, gsems, ssems):
        start = pl.multiple_of(inp_smem[1] - SEQ_LEN, 8)
        g = [None] * NCHUNKS
        s = [None] * NCHUNKS
        for i in range(NCHUNKS + DEPTH):
            if i < NCHUNKS:
                b = i % NBUF
                if i >= NBUF:
                    s[i - NBUF].wait()
                g[i] = pltpu.make_async_copy(
                    enc_hbm.at[pl.ds(start + i * CHUNK, CHUNK)],
                    buf.at[b],
                    gsems.at[b],
                )
                g[i].start()
            j = i - DEPTH
            if 0 <= j < NCHUNKS:
                jb = j % NBUF
                g[j].wait()
                s[j] = pltpu.make_async_copy(
                    buf.at[jb], out_hbm.at[pl.ds(j * CHUNK, CHUNK)], ssems.at[jb]
                )
                s[j].start()
        for j in range(max(0, NCHUNKS - NBUF), NCHUNKS):
            s[j].wait()

    return pl.pallas_call(
        body,
        in_specs=[
            pl.BlockSpec(memory_space=pltpu.MemorySpace.SMEM),
            pl.BlockSpec(memory_space=pltpu.MemorySpace.HBM),
        ],
        out_specs=pl.BlockSpec(memory_space=pltpu.MemorySpace.HBM),
        out_shape=jax.ShapeDtypeStruct((SEQ_LEN, EMB), jnp.float32),
        scratch_shapes=[
            pltpu.VMEM((NBUF, CHUNK, EMB), jnp.float32),
            pltpu.SemaphoreType.DMA((NBUF,)),
            pltpu.SemaphoreType.DMA((NBUF,)),
        ],
    )(input, encoding)


# speedup vs baseline: 1.0101x; 1.0101x over previous
import jax
import jax.numpy as jnp
from jax.experimental import pallas as pl
from jax.experimental.pallas import tpu as pltpu

SEQ_LEN = 4096
EMB = 1024
CHUNK = 512
NBUF = 8
DEPTH = 8
NCHUNKS = SEQ_LEN // CHUNK


def kernel(input, encoding):
    def body(inp_smem, enc_hbm, out_hbm, buf, gsems, ssems):
        start = pl.multiple_of(inp_smem[1] - SEQ_LEN, 8)
        g = [None] * NCHUNKS
        s = [None] * NCHUNKS
        for i in range(NCHUNKS + DEPTH):
            if i < NCHUNKS:
                b = i % NBUF
                if i >= NBUF:
                    s[i - NBUF].wait()
                g[i] = pltpu.make_async_copy(
                    enc_hbm.at[pl.ds(start + i * CHUNK, CHUNK)],
                    buf.at[b],
                    gsems.at[b],
                )
                g[i].start()
            j = i - DEPTH
            if 0 <= j < NCHUNKS:
                jb = j % NBUF
                g[j].wait()
                s[j] = pltpu.make_async_copy(
                    buf.at[jb], out_hbm.at[pl.ds(j * CHUNK, CHUNK)], ssems.at[jb]
                )
                s[j].start()
        for j in range(max(0, NCHUNKS - NBUF), NCHUNKS):
            s[j].wait()

    return pl.pallas_call(
        body,
        in_specs=[
            pl.BlockSpec(memory_space=pltpu.MemorySpace.SMEM),
            pl.BlockSpec(memory_space=pltpu.MemorySpace.HBM),
        ],
        out_specs=pl.BlockSpec(memory_space=pltpu.MemorySpace.HBM),
        out_shape=jax.ShapeDtypeStruct((SEQ_LEN, EMB), jnp.float32),
        scratch_shapes=[
            pltpu.VMEM((NBUF, CHUNK, EMB), jnp.float32),
            pltpu.SemaphoreType.DMA((NBUF,)),
            pltpu.SemaphoreType.DMA((NBUF,)),
        ],
    )(input, encoding)


# speedup vs baseline: 1.0165x; 1.0063x over previous
import jax
import jax.numpy as jnp
from jax.experimental import pallas as pl
from jax.experimental.pallas import tpu as pltpu

SEQ_LEN = 4096
EMB = 1024
CHUNK = 1024
NBUF = 4
DEPTH = 4
NCHUNKS = SEQ_LEN // CHUNK


def kernel(input, encoding):
    def body(inp_smem, enc_hbm, out_hbm, buf, gsems, ssems):
        start = pl.multiple_of(inp_smem[1] - SEQ_LEN, 8)
        g = [None] * NCHUNKS
        s = [None] * NCHUNKS
        for i in range(NCHUNKS + DEPTH):
            if i < NCHUNKS:
                b = i % NBUF
                if i >= NBUF:
                    s[i - NBUF].wait()
                g[i] = pltpu.make_async_copy(
                    enc_hbm.at[pl.ds(start + i * CHUNK, CHUNK)],
                    buf.at[b],
                    gsems.at[b],
                )
                g[i].start()
            j = i - DEPTH
            if 0 <= j < NCHUNKS:
                jb = j % NBUF
                g[j].wait()
                s[j] = pltpu.make_async_copy(
                    buf.at[jb], out_hbm.at[pl.ds(j * CHUNK, CHUNK)], ssems.at[jb]
                )
                s[j].start()
        for j in range(max(0, NCHUNKS - NBUF), NCHUNKS):
            s[j].wait()

    return pl.pallas_call(
        body,
        in_specs=[
            pl.BlockSpec(memory_space=pltpu.MemorySpace.SMEM),
            pl.BlockSpec(memory_space=pltpu.MemorySpace.HBM),
        ],
        out_specs=pl.BlockSpec(memory_space=pltpu.MemorySpace.HBM),
        out_shape=jax.ShapeDtypeStruct((SEQ_LEN, EMB), jnp.float32),
        scratch_shapes=[
            pltpu.VMEM((NBUF, CHUNK, EMB), jnp.float32),
            pltpu.SemaphoreType.DMA((NBUF,)),
            pltpu.SemaphoreType.DMA((NBUF,)),
        ],
    )(input, encoding)
